# baseline (speedup 1.0000x reference)
import jax
import jax.numpy as jnp
from jax import lax
from jax.experimental import pallas as pl
from jax.experimental.pallas import tpu as pltpu

_NBUF = 3
_TAIL = 128


def _chunk_plan(B, S):
    plan = [(b, 0, S) for b in range(B - 1)]
    plan.append((B - 1, 0, S - _TAIL))
    plan.append((B - 1, S - _TAIL, _TAIL))
    return plan


def _attn_body(idx_ref, h_ref, o_ref, bufs, rows, sacc, csem, rsem):
    B, S, D = h_ref.shape
    K2 = idx_ref.shape[-1]
    K = K2 // 2
    plan = _chunk_plan(B, S)
    nchunks = len(plan)

    def chunk_copy(i):
        b, r0, n = plan[i]
        return pltpu.make_async_copy(
            h_ref.at[b, pl.ds(r0, n), :], bufs.at[i % _NBUF, pl.ds(0, n), :],
            csem.at[i % _NBUF])

    def row_copies(b):
        hs = []
        for g in range(K2):
            hs.append(pltpu.make_async_copy(
                h_ref.at[b, idx_ref[b, g]], rows.at[b * K2 + g], rsem))
        return hs

    row_handles = {0: row_copies(0)}
    for h in row_handles[0]:
        h.start()
    handles = []
    for i in range(min(_NBUF, nchunks)):
        handles.append(chunk_copy(i))
        handles[i].start()

    e12 = None
    for i in range(nchunks):
        b, r0, n = plan[i]
        if r0 == 0:
            for h in row_handles[b]:
                h.wait()
            e1 = rows[b * K2, :]
            e2 = rows[b * K2 + K, :]
            for k in range(1, K):
                e1 = e1 + rows[b * K2 + k, :]
                e2 = e2 + rows[b * K2 + K + k, :]
            e12 = jnp.stack([e1, e2], axis=0)
            if b + 1 < B:
                row_handles[b + 1] = row_copies(b + 1)
                for h in row_handles[b + 1]:
                    h.start()
        handles[i].wait()
        s = lax.dot_general(
            bufs[i % _NBUF, :n, :], e12, (((1,), (1,)), ((), ())),
            preferred_element_type=jnp.float32,
        )
        sacc[pl.ds(r0, n), :] = s
        if i + _NBUF < nchunks:
            handles.append(chunk_copy(i + _NBUF))
            handles[i + _NBUF].start()
        if r0 + n == S:
            t = sacc[...]
            p = jnp.exp(t - jnp.max(t, axis=0, keepdims=True))
            w = p / jnp.sum(p, axis=0, keepdims=True)
            o_ref[b, :] = 0.5 * jnp.sum(w, axis=1)


def kernel(input_embed_M, e1_index, e2_index):
    B, S, D = input_embed_M.shape
    K = e1_index.shape[-1]
    eidx = jnp.concatenate(
        [e1_index.astype(jnp.int32), e2_index.astype(jnp.int32)], axis=1
    )
    return pl.pallas_call(
        _attn_body,
        in_specs=[
            pl.BlockSpec(memory_space=pltpu.SMEM),
            pl.BlockSpec(memory_space=pltpu.MemorySpace.HBM),
        ],
        out_specs=pl.BlockSpec(memory_space=pltpu.VMEM),
        out_shape=jax.ShapeDtypeStruct((B, S), jnp.float32),
        scratch_shapes=[
            pltpu.VMEM((_NBUF, S, D), jnp.float32),
            pltpu.VMEM((B * 2 * K, D), jnp.float32),
            pltpu.VMEM((S, 2), jnp.float32),
            pltpu.SemaphoreType.DMA((_NBUF,)),
            pltpu.SemaphoreType.DMA,
        ],
    )(eidx, input_embed_M)


# speedup vs baseline: 1.2532x; 1.2532x over previous
import functools

import jax
import jax.numpy as jnp
from jax import lax
from jax.experimental import pallas as pl
from jax.experimental.pallas import tpu as pltpu


def _attn_body(idx_ref, h_ref, o_ref):
    K = idx_ref.shape[-1] // 2
    hb = h_ref[0]
    e1 = h_ref[0, idx_ref[0, 0, 0], :]
    e2 = h_ref[0, idx_ref[0, 0, K], :]
    for k in range(1, K):
        e1 = e1 + h_ref[0, idx_ref[0, 0, k], :]
        e2 = e2 + h_ref[0, idx_ref[0, 0, K + k], :]
    e12 = jnp.stack([e1, e2], axis=0)
    s = lax.dot_general(
        hb, e12, (((1,), (1,)), ((), ())),
        preferred_element_type=jnp.float32,
    )
    p = jnp.exp(s - jnp.max(s, axis=0, keepdims=True))
    w = p / jnp.sum(p, axis=0, keepdims=True)
    o_ref[0, 0] = 0.5 * jnp.sum(w, axis=1)


def kernel(input_embed_M, e1_index, e2_index):
    B, S, D = input_embed_M.shape
    K = e1_index.shape[-1]
    eidx = jnp.concatenate(
        [e1_index.astype(jnp.int32), e2_index.astype(jnp.int32)], axis=1
    ).reshape(B, 1, 2 * K)
    out = pl.pallas_call(
        _attn_body,
        grid=(B,),
        in_specs=[
            pl.BlockSpec((1, 1, 2 * K), lambda b: (b, 0, 0), memory_space=pltpu.SMEM),
            pl.BlockSpec((1, S, D), lambda b: (b, 0, 0)),
        ],
        out_specs=pl.BlockSpec((1, 1, S), lambda b: (b, 0, 0)),
        out_shape=jax.ShapeDtypeStruct((B, 1, S), jnp.float32),
    )(eidx, input_embed_M)
    return out[:, 0, :]
